# speedup vs baseline: 1.5079x; 1.5079x over previous
import jax
import jax.numpy as jnp
from jax.experimental import pallas as pl

_NSEG = 64
_D = 45


def _mlp(x, w0, b0, w1, b1, w2, b2, w3, b3):
    h = jnp.maximum(jnp.dot(x, w0, preferred_element_type=jnp.float32) + b0, 0.0)
    h = jnp.maximum(jnp.dot(h, w1, preferred_element_type=jnp.float32) + b1, 0.0)
    h = jnp.maximum(jnp.dot(h, w2, preferred_element_type=jnp.float32) + b2, 0.0)
    return jnp.dot(h, w3, preferred_element_type=jnp.float32) + b3


def _pair_kernel(c_ref, x_ref, ids_ref, w0_ref, b0_ref, w1_ref, b1_ref,
                 w2_ref, b2_ref, w3_ref, b3_ref, out_ref):
    i = pl.program_id(0)
    args = (w0_ref[...], b0_ref[...], w1_ref[...], b1_ref[...],
            w2_ref[...], b2_ref[...], w3_ref[...], b3_ref[...])
    sc = _mlp(c_ref[...], *args)
    sx = _mlp(x_ref[...], *args)
    d = sc - sx
    ids = ids_ref[0]
    tile = d.shape[0]
    seg = jax.lax.broadcasted_iota(jnp.int32, (tile, _NSEG), 1)
    contrib = jnp.where(ids == seg, d, 0.0)
    partial = jnp.sum(contrib, axis=0, keepdims=True)

    @pl.when(i == 0)
    def _():
        out_ref[...] = jnp.zeros_like(out_ref)

    out_ref[...] += partial


def _run_pair(cx, row_offset, x, ids, tile, ws):
    n = x.shape[0]
    nsteps = n // tile
    off = row_offset // tile
    ids3 = ids.reshape(nsteps, tile, 1)
    w0, b0, w1, b1, w2, b2, w3, b3 = ws
    wspec = lambda a: pl.BlockSpec(a.shape, lambda i: (0,) * a.ndim)
    return pl.pallas_call(
        _pair_kernel,
        grid=(nsteps,),
        in_specs=[
            pl.BlockSpec((tile, _D), lambda i, off=off: (off + i, 0)),
            pl.BlockSpec((tile, _D), lambda i: (i, 0)),
            pl.BlockSpec((1, tile, 1), lambda i: (i, 0, 0)),
            wspec(w0), wspec(b0), wspec(w1), wspec(b1),
            wspec(w2), wspec(b2), wspec(w3), wspec(b3),
        ],
        out_specs=pl.BlockSpec((1, _NSEG), lambda i: (0, 0)),
        out_shape=jax.ShapeDtypeStruct((1, _NSEG), jnp.float32),
    )(cx, x, ids3, w0, b0, w1, b1, w2, b2, w3, b3)


def kernel(protein_conv_out, ligand_conv_out, complex_conv_out,
           protein_segment_ids, ligand_segment_ids,
           W0, b0, W1, b1, W2, b2, W3, b3):
    v1 = protein_conv_out.shape[0]
    ws = (W0, b0.reshape(1, -1), W1, b1.reshape(1, -1),
          W2, b2.reshape(1, -1), W3, b3.reshape(1, -1))
    pa = _run_pair(complex_conv_out, 0, protein_conv_out,
                   protein_segment_ids, 2000, ws)
    pb = _run_pair(complex_conv_out, v1, ligand_conv_out,
                   ligand_segment_ids, 2000, ws)
    return (pa + pb).reshape(_NSEG, 1)


# speedup vs baseline: 1.7677x; 1.1723x over previous
import jax
import jax.numpy as jnp
from jax.experimental import pallas as pl

_NSEG = 64
_D = 45
_P = 8


def _mlp(x, w0, b0, w1, b1, w2, b2, w3, b3):
    h = jnp.maximum(jnp.dot(x, w0, preferred_element_type=jnp.float32) + b0, 0.0)
    h = jnp.maximum(jnp.dot(h, w1, preferred_element_type=jnp.float32) + b1, 0.0)
    h = jnp.maximum(jnp.dot(h, w2, preferred_element_type=jnp.float32) + b2, 0.0)
    return jnp.dot(h, w3, preferred_element_type=jnp.float32) + b3


def _pair_kernel(c_ref, x_ref, ids_ref, w0_ref, b0_ref, w1_ref, b1_ref,
                 w2_ref, b2_ref, w3_ref, b3_ref, out_ref):
    i = pl.program_id(0)
    args = (w0_ref[...], b0_ref[...], w1_ref[...], b1_ref[...],
            w2_ref[...], b2_ref[...], w3_ref[...], b3_ref[...])
    sc = _mlp(c_ref[0], *args)
    sx = _mlp(x_ref[0], *args)
    d = sc - sx
    ids = ids_ref[0]
    tile_p = d.shape[0]
    seg = jax.lax.broadcasted_iota(jnp.int32, (tile_p, _NSEG), 1)
    acc = jnp.zeros((tile_p, _NSEG), jnp.float32)
    for p in range(_P):
        acc += jnp.where(ids[:, p:p + 1] == seg, d[:, p:p + 1], 0.0)
    partial = jnp.sum(acc, axis=0, keepdims=True)

    @pl.when(i == 0)
    def _():
        out_ref[...] = jnp.zeros_like(out_ref)

    out_ref[...] += partial


def _run_pair(cx, row_offset, x, ids, tile, ws):
    n = x.shape[0]
    nsteps = n // tile
    tile_p = tile // _P
    wide = _P * _D
    off = row_offset // tile
    cx3 = cx.reshape(cx.shape[0] // tile, tile_p, wide)
    x3 = x.reshape(nsteps, tile_p, wide)
    ids3 = ids.reshape(nsteps, tile_p, _P)
    w0, b0, w1, b1, w2, b2, w3, b3 = ws
    wspec = lambda a: pl.BlockSpec(a.shape, lambda i: (0,) * a.ndim)
    return pl.pallas_call(
        _pair_kernel,
        grid=(nsteps,),
        in_specs=[
            pl.BlockSpec((1, tile_p, wide), lambda i, off=off: (off + i, 0, 0)),
            pl.BlockSpec((1, tile_p, wide), lambda i: (i, 0, 0)),
            pl.BlockSpec((1, tile_p, _P), lambda i: (i, 0, 0)),
            wspec(w0), wspec(b0), wspec(w1), wspec(b1),
            wspec(w2), wspec(b2), wspec(w3), wspec(b3),
        ],
        out_specs=pl.BlockSpec((1, _NSEG), lambda i: (0, 0)),
        out_shape=jax.ShapeDtypeStruct((1, _NSEG), jnp.float32),
    )(cx3, x3, ids3, w0, b0, w1, b1, w2, b2, w3, b3)


def kernel(protein_conv_out, ligand_conv_out, complex_conv_out,
           protein_segment_ids, ligand_segment_ids,
           W0, b0, W1, b1, W2, b2, W3, b3):
    v1 = protein_conv_out.shape[0]
    eye = jnp.eye(_P, dtype=jnp.float32)
    ws = (jnp.kron(eye, W0), jnp.tile(b0, _P).reshape(1, -1),
          jnp.kron(eye, W1), jnp.tile(b1, _P).reshape(1, -1),
          jnp.kron(eye, W2), jnp.tile(b2, _P).reshape(1, -1),
          jnp.kron(eye, W3), jnp.tile(b3, _P).reshape(1, -1))
    pa = _run_pair(complex_conv_out, 0, protein_conv_out,
                   protein_segment_ids, 2000, ws)
    pb = _run_pair(complex_conv_out, v1, ligand_conv_out,
                   ligand_segment_ids, 2000, ws)
    return (pa + pb).reshape(_NSEG, 1)
